# baseline (speedup 1.0000x reference)
import jax
import jax.numpy as jnp
from jax import lax
from jax.experimental import pallas as pl
from jax.experimental.pallas import tpu as pltpu

B_USERS = 256
B_ITEMS = 256
HIDDEN_DIM = 128
N_ROWS = 1024
CHUNK = 256
N_CHUNKS = N_ROWS // CHUNK


def _body(uid_ref, iid_ref, utab_hbm, itab_hbm, o_ref,
          utab_v, itab_v, sem_u, sem_i):
  copies_u = []
  copies_i = []
  for c in range(N_CHUNKS):
    sl = pl.ds(c * CHUNK, CHUNK)
    cu = pltpu.make_async_copy(utab_hbm.at[sl, :], utab_v.at[sl, :],
                               sem_u.at[c])
    ci = pltpu.make_async_copy(itab_hbm.at[sl, :], itab_v.at[sl, :],
                               sem_i.at[c])
    cu.start()
    ci.start()
    copies_u.append(cu)
    copies_i.append(ci)

  uid = uid_ref[0]
  iid = iid_ref[0]

  u = None
  v = None
  for c in range(N_CHUNKS):
    sl = pl.ds(c * CHUNK, CHUNK)
    rows = c * CHUNK + lax.broadcasted_iota(jnp.int32, (B_USERS, CHUNK), 1)
    pu = (uid[:, None] == rows).astype(jnp.float32)
    pv = (iid[:, None] == rows).astype(jnp.float32)
    copies_u[c].wait()
    du = jnp.dot(pu, utab_v[sl, :], preferred_element_type=jnp.float32)
    copies_i[c].wait()
    dv = jnp.dot(pv, itab_v[sl, :], preferred_element_type=jnp.float32)
    u = du if u is None else u + du
    v = dv if v is None else v + dv

  o_ref[...] = lax.dot_general(
      u, v, dimension_numbers=(((1,), (1,)), ((), ())),
      preferred_element_type=jnp.float32)


_call = pl.pallas_call(
    _body,
    in_specs=[
        pl.BlockSpec((1, B_USERS), lambda: (0, 0)),
        pl.BlockSpec((1, B_ITEMS), lambda: (0, 0)),
        pl.BlockSpec(memory_space=pl.ANY),
        pl.BlockSpec(memory_space=pl.ANY),
    ],
    out_specs=pl.BlockSpec((B_USERS, B_ITEMS), lambda: (0, 0)),
    out_shape=jax.ShapeDtypeStruct((B_USERS, B_ITEMS), jnp.float32),
    scratch_shapes=[
        pltpu.VMEM((N_ROWS, HIDDEN_DIM), jnp.float32),
        pltpu.VMEM((N_ROWS, HIDDEN_DIM), jnp.float32),
        pltpu.SemaphoreType.DMA((N_CHUNKS,)),
        pltpu.SemaphoreType.DMA((N_CHUNKS,)),
    ],
)


@jax.jit
def kernel(user_ids, item_ids, user_table, item_table):
  return _call(user_ids.reshape(1, B_USERS), item_ids.reshape(1, B_ITEMS),
               user_table, item_table)
